# baseline (speedup 1.0000x reference)
import functools

import jax
import jax.numpy as jnp
from jax import lax
from jax.experimental import pallas as pl
from jax.experimental.pallas import tpu as pltpu
from jax.experimental.pallas import tpu_sc as plsc

N_CORE = 20000
N_AUX = 5000
N_BINS = N_CORE + N_AUX
BINS_PAD = 25088
D = 256
N_IDS = 1024000
NUM_WORKERS = 32
IDS_PER_W = N_IDS // NUM_WORKERS
LANES = 16

_mesh = plsc.VectorSubcoreMesh(core_axis_name="c", subcore_axis_name="s")


@functools.partial(
    pl.kernel,
    out_type=jax.ShapeDtypeStruct((NUM_WORKERS, BINS_PAD), jnp.float32),
    mesh=_mesh,
    scratch_types=[
        pltpu.VMEM((IDS_PER_W,), jnp.int32),
        pltpu.VMEM((BINS_PAD,), jnp.float32),
    ],
    compiler_params=pltpu.CompilerParams(needs_layout_passes=False),
)
def _histogram(ids_hbm, out_hbm, ids_v, counts_v):
    wid = lax.axis_index("s") * 2 + lax.axis_index("c")
    base = wid * IDS_PER_W
    pltpu.sync_copy(ids_hbm.at[pl.ds(base, IDS_PER_W)], ids_v)

    zeros = jnp.zeros((LANES,), jnp.float32)

    def zero_body(i, carry):
        counts_v[pl.ds(i * LANES, LANES)] = zeros
        return carry

    lax.fori_loop(0, BINS_PAD // LANES, zero_body, 0)

    ones = jnp.ones((LANES,), jnp.float32)

    def scat_body(i, carry):
        idx = ids_v[pl.ds(i * LANES, LANES)]
        plsc.addupdate_scatter(counts_v, [idx], ones)
        return carry

    lax.fori_loop(0, IDS_PER_W // LANES, scat_body, 0)

    pltpu.sync_copy(counts_v, out_hbm.at[wid])


_CB = 4000
_AB = 1000
_STEPS = 5


def _matvec_body(cc_ref, ca_ref, core_ref, aux_ref, out_ref):
    j = pl.program_id(0)

    @pl.when(j == 0)
    def _init():
        out_ref[...] = jnp.zeros_like(out_ref)

    cc = jnp.sum(cc_ref[...], axis=1, keepdims=True)
    ca = jnp.sum(ca_ref[...], axis=1, keepdims=True)
    acc = lax.dot_general(
        cc, core_ref[...], (((0,), (0,)), ((), ())),
        precision=lax.Precision.HIGHEST,
        preferred_element_type=jnp.float32,
    )
    acc = acc + lax.dot_general(
        ca, aux_ref[...], (((0,), (0,)), ((), ())),
        precision=lax.Precision.HIGHEST,
        preferred_element_type=jnp.float32,
    )
    out_ref[...] += acc

    @pl.when(j == _STEPS - 1)
    def _fin():
        out_ref[...] *= (1.0 / N_IDS)


def _weighted_sum(counts_t, core_feats, aux_feats):
    return pl.pallas_call(
        _matvec_body,
        grid=(_STEPS,),
        in_specs=[
            pl.BlockSpec((_CB, NUM_WORKERS), lambda j: (j, 0)),
            pl.BlockSpec((_AB, NUM_WORKERS), lambda j: (N_CORE // _AB + j, 0)),
            pl.BlockSpec((_CB, D), lambda j: (j, 0)),
            pl.BlockSpec((_AB, D), lambda j: (j, 0)),
        ],
        out_specs=pl.BlockSpec((1, D), lambda j: (0, 0)),
        out_shape=jax.ShapeDtypeStruct((1, D), jnp.float32),
    )(counts_t, counts_t, core_feats, aux_feats)


def kernel(core_feats, aux_feats, id_map):
    ids = id_map.reshape(-1).astype(jnp.int32)
    counts = _histogram(ids)
    counts_t = counts.T
    return _weighted_sum(counts_t, core_feats, aux_feats)


# speedup vs baseline: 1.1719x; 1.1719x over previous
import functools

import jax
import jax.numpy as jnp
from jax import lax
from jax.experimental import pallas as pl
from jax.experimental.pallas import tpu as pltpu
from jax.experimental.pallas import tpu_sc as plsc

N_CORE = 20000
N_AUX = 5000
N_BINS = N_CORE + N_AUX
BINS_PAD = 25088
D = 256
N_IDS = 1024000
NUM_WORKERS = 32
IDS_PER_W = N_IDS // NUM_WORKERS
LANES = 16

_mesh = plsc.VectorSubcoreMesh(core_axis_name="c", subcore_axis_name="s")


@functools.partial(
    pl.kernel,
    out_type=jax.ShapeDtypeStruct((NUM_WORKERS, BINS_PAD), jnp.float32),
    mesh=_mesh,
    scratch_types=[
        pltpu.VMEM((IDS_PER_W,), jnp.int32),
        pltpu.VMEM((BINS_PAD,), jnp.float32),
    ],
    compiler_params=pltpu.CompilerParams(needs_layout_passes=False),
)
def _histogram(ids_hbm, out_hbm, ids_v, counts_v):
    wid = lax.axis_index("s") * 2 + lax.axis_index("c")
    base = wid * IDS_PER_W
    pltpu.sync_copy(ids_hbm.at[pl.ds(base, IDS_PER_W)], ids_v)

    zeros = jnp.zeros((LANES,), jnp.float32)

    @plsc.parallel_loop(0, BINS_PAD // LANES, unroll=8)
    def _zero(i):
        counts_v[pl.ds(i * LANES, LANES)] = zeros

    ones = jnp.ones((LANES,), jnp.float32)

    @plsc.parallel_loop(0, IDS_PER_W // LANES, unroll=8)
    def _scat(i):
        idx = ids_v[pl.ds(i * LANES, LANES)]
        plsc.addupdate_scatter(counts_v, [idx], ones)

    pltpu.sync_copy(counts_v, out_hbm.at[wid])


_CB = 4000
_AB = 1000
_STEPS = 5


def _matvec_body(cc_ref, ca_ref, core_ref, aux_ref, out_ref):
    j = pl.program_id(0)

    @pl.when(j == 0)
    def _init():
        out_ref[...] = jnp.zeros_like(out_ref)

    cc = jnp.sum(cc_ref[...], axis=1, keepdims=True)
    ca = jnp.sum(ca_ref[...], axis=1, keepdims=True)
    acc = lax.dot_general(
        cc, core_ref[...], (((0,), (0,)), ((), ())),
        precision=lax.Precision.HIGHEST,
        preferred_element_type=jnp.float32,
    )
    acc = acc + lax.dot_general(
        ca, aux_ref[...], (((0,), (0,)), ((), ())),
        precision=lax.Precision.HIGHEST,
        preferred_element_type=jnp.float32,
    )
    out_ref[...] += acc

    @pl.when(j == _STEPS - 1)
    def _fin():
        out_ref[...] *= (1.0 / N_IDS)


def _weighted_sum(counts_t, core_feats, aux_feats):
    return pl.pallas_call(
        _matvec_body,
        grid=(_STEPS,),
        in_specs=[
            pl.BlockSpec((_CB, NUM_WORKERS), lambda j: (j, 0)),
            pl.BlockSpec((_AB, NUM_WORKERS), lambda j: (N_CORE // _AB + j, 0)),
            pl.BlockSpec((_CB, D), lambda j: (j, 0)),
            pl.BlockSpec((_AB, D), lambda j: (j, 0)),
        ],
        out_specs=pl.BlockSpec((1, D), lambda j: (0, 0)),
        out_shape=jax.ShapeDtypeStruct((1, D), jnp.float32),
    )(counts_t, counts_t, core_feats, aux_feats)


def kernel(core_feats, aux_feats, id_map):
    ids = id_map.reshape(-1).astype(jnp.int32)
    counts = _histogram(ids)
    counts_t = counts.T
    return _weighted_sum(counts_t, core_feats, aux_feats)
